# baseline (speedup 1.0000x reference)
import functools

import jax
import jax.numpy as jnp
from jax.experimental import pallas as pl
from jax.experimental.pallas import tpu as pltpu

_Z_CELL = 0.1
_Z_HID = 0.1
_TS = 8


def _cell(xw, h, c, hid):
    gi = jax.nn.sigmoid(xw[:, :hid])
    gf = jax.nn.sigmoid(xw[:, hid : 2 * hid])
    gg = jnp.tanh(xw[:, 2 * hid : 3 * hid])
    go = jax.nn.sigmoid(xw[:, 3 * hid :])
    c_new = gf * c + gi * gg
    h_new = go * jnp.tanh(c_new)
    c_out = (1.0 - _Z_CELL) * c_new + _Z_CELL * c
    h_out = (1.0 - _Z_HID) * h_new + _Z_HID * h
    return h_out, c_out


def _fused_kernel(
    xf_ref,
    xb_ref,
    wih_ref,
    whh_ref,
    bias_ref,
    of_ref,
    ob_ref,
    xw_ref,
    h_ref,
    c_ref,
    *,
    ts,
    hid,
    nb,
):
    t = pl.program_id(0)
    rows = ts * nb

    @pl.when(t == 1)
    def _():
        h_ref[...] = jnp.zeros_like(h_ref)
        c_ref[...] = jnp.zeros_like(c_ref)

    buf = xw_ref.at[(t + 1) % 2]
    for j in range(ts):
        jb = ts - 1 - j
        h = h_ref[...]
        gates = jnp.dot(
            h.astype(jnp.bfloat16), whh_ref[...], preferred_element_type=jnp.float32
        )
        hf, cf = _cell(gates[:nb] + buf[j * nb : (j + 1) * nb], h[:nb], c_ref[:nb], hid)
        hb, cb = _cell(
            gates[nb:] + buf[rows + jb * nb : rows + (jb + 1) * nb],
            h[nb:],
            c_ref[nb:],
            hid,
        )
        h_ref[:nb] = hf
        h_ref[nb:] = hb
        c_ref[:nb] = cf
        c_ref[nb:] = cb
        of_ref[j] = hf
        ob_ref[jb] = hb

    xcat = jnp.concatenate(
        [xf_ref[...].reshape(rows, -1), xb_ref[...].reshape(rows, -1)], axis=0
    ).astype(jnp.bfloat16)
    xwp = (
        jnp.dot(xcat, wih_ref[...], preferred_element_type=jnp.float32)
        + bias_ref[...]
    )
    par = t % 2
    G = 4 * hid
    for k in range(0, G, 512):
        xw_ref[par, :, k : k + 512] = xwp[:, k : k + 512]


def kernel(inputs, W_ih, W_hh, b_ih, b_hh):
    T, B, I = inputs.shape
    G = W_ih.shape[0]
    hid = G // 4
    ts = _TS
    nt = T // ts

    bias = (b_ih + b_hh).reshape(1, G)

    out_f, out_b = pl.pallas_call(
        functools.partial(_fused_kernel, ts=ts, hid=hid, nb=B),
        grid=(nt + 1,),
        in_specs=[
            pl.BlockSpec((ts, B, I), lambda t: (jnp.minimum(t, nt - 1), 0, 0)),
            pl.BlockSpec((ts, B, I), lambda t: (jnp.maximum(nt - 1 - t, 0), 0, 0)),
            pl.BlockSpec((I, G), lambda t: (0, 0)),
            pl.BlockSpec((hid, G), lambda t: (0, 0)),
            pl.BlockSpec((1, G), lambda t: (0, 0)),
        ],
        out_specs=[
            pl.BlockSpec((ts, B, hid), lambda t: (jnp.maximum(t - 1, 0), 0, 0)),
            pl.BlockSpec((ts, B, hid), lambda t: (jnp.minimum(nt - t, nt - 1), 0, 0)),
        ],
        out_shape=[
            jax.ShapeDtypeStruct((T, B, hid), jnp.float32),
            jax.ShapeDtypeStruct((T, B, hid), jnp.float32),
        ],
        scratch_shapes=[
            pltpu.VMEM((2, 2 * ts * B, G), jnp.float32),
            pltpu.VMEM((2 * B, hid), jnp.float32),
            pltpu.VMEM((2 * B, hid), jnp.float32),
        ],
        compiler_params=pltpu.CompilerParams(
            dimension_semantics=("arbitrary",),
            vmem_limit_bytes=48 * 1024 * 1024,
        ),
        name="lstm_fused",
    )(
        inputs,
        inputs,
        W_ih.T.astype(jnp.bfloat16),
        W_hh.T.astype(jnp.bfloat16),
        bias,
    )

    return out_f + out_b
